# Initial kernel scaffold; baseline (speedup 1.0000x reference)
import functools
import math

import jax
import jax.numpy as jnp
from jax import lax
from jax.experimental import pallas as pl
from jax.experimental.pallas import tpu as pltpu
from jax.experimental.pallas import tpu_sc as plsc

_NC = 2
_NS = 16
_NW = _NC * _NS
_LANES = 16
_WIN = 128

_LAMDA = 0.5


@functools.lru_cache(maxsize=None)
def _make_prop(n, e, h):
    assert e % _WIN == 0
    nrows = e // _WIN
    base_rows = nrows // _NW
    rem = nrows % _NW
    rps = n // _NS
    assert n % _NS == 0
    ncol = h // _LANES

    mesh = plsc.VectorSubcoreMesh(core_axis_name="c", subcore_axis_name="s")

    def body(h_hbm, src_hbm, dst_hbm, norm_hbm, zer_hbm, out_hbm,
             acc, src_v, dst_v, norm_v, rows_v):
        cid = lax.axis_index("c")
        sid = lax.axis_index("s")
        wid = sid * _NC + cid

        pltpu.sync_copy(zer_hbm, acc.at[pl.ds(sid * rps, rps)])
        plsc.subcore_barrier()

        nwin = jnp.where(wid < rem, base_rows + 1, base_rows)

        def window(j, carry):
            row = wid + _NW * j
            pltpu.sync_copy(src_hbm.at[row], src_v)
            pltpu.sync_copy(dst_hbm.at[row], dst_v)
            pltpu.sync_copy(norm_hbm.at[row], norm_v)
            pltpu.sync_copy(h_hbm.at[src_v], rows_v)

            def scale(ei, c2):
                nv = norm_v[ei]
                vb = jnp.full((_LANES,), nv, jnp.float32)
                for cc in range(ncol):
                    sl = pl.ds(cc * _LANES, _LANES)
                    rows_v[ei, sl] = rows_v[ei, sl] * vb
                return c2

            lax.fori_loop(0, _WIN, scale, 0, unroll=2)
            pltpu.sync_copy(rows_v, acc.at[dst_v], add=True)
            return carry

        lax.fori_loop(0, nwin, window, 0)
        plsc.subcore_barrier()
        pltpu.sync_copy(acc.at[pl.ds(sid * rps, rps)],
                        out_hbm.at[pl.ds(cid * n + sid * rps, rps)])

    return pl.kernel(
        body,
        out_type=jax.ShapeDtypeStruct((2 * n, h), jnp.float32),
        mesh=mesh,
        scratch_types=[
            pltpu.VMEM_SHARED((n, h), jnp.float32),
            pltpu.VMEM((_WIN,), jnp.int32),
            pltpu.VMEM((_WIN,), jnp.int32),
            pltpu.VMEM((_WIN,), jnp.float32),
            pltpu.VMEM((_WIN, h), jnp.float32),
        ],
    )


_BLK = 400


def _fc0(features, w, b):
    n, din = features.shape
    hdim = w.shape[1]

    def bdy(x_ref, w_ref, b_ref, o_ref):
        o_ref[...] = jnp.maximum(
            jnp.dot(x_ref[...], w_ref[...], preferred_element_type=jnp.float32)
            + b_ref[...], 0.0)

    return pl.pallas_call(
        bdy,
        grid=(n // _BLK,),
        in_specs=[
            pl.BlockSpec((_BLK, din), lambda i: (i, 0)),
            pl.BlockSpec((din, hdim), lambda i: (0, 0)),
            pl.BlockSpec((1, hdim), lambda i: (0, 0)),
        ],
        out_specs=pl.BlockSpec((_BLK, hdim), lambda i: (i, 0)),
        out_shape=jax.ShapeDtypeStruct((n, hdim), jnp.float32),
    )(features, w, b.reshape(1, hdim))


def _layer(a, h0, pp, prev, w, b, *, beta, tmul, pmul, dorelu):
    n, hdim = h0.shape

    def bdy(a_ref, h0_ref, p0_ref, p1_ref, pv_ref, w_ref, b_ref, o_ref):
        av = a_ref[0]
        tx = tmul * (p0_ref[...] + p1_ref[...]) - pmul * pv_ref[...]
        hi = av * h0_ref[...] + (1.0 - av) * tx
        x = ((1.0 - beta) * hi
             + beta * jnp.dot(hi, w_ref[...], preferred_element_type=jnp.float32)
             + b_ref[...])
        o_ref[...] = jnp.maximum(x, 0.0) if dorelu else x

    nblk = n // _BLK
    return pl.pallas_call(
        bdy,
        grid=(nblk,),
        in_specs=[
            pl.BlockSpec(memory_space=pltpu.SMEM),
            pl.BlockSpec((_BLK, hdim), lambda i: (i, 0)),
            pl.BlockSpec((_BLK, hdim), lambda i: (i, 0)),
            pl.BlockSpec((_BLK, hdim), lambda i, _n=nblk: (i + _n, 0)),
            pl.BlockSpec((_BLK, hdim), lambda i: (i, 0)),
            pl.BlockSpec((hdim, hdim), lambda i: (0, 0)),
            pl.BlockSpec((1, hdim), lambda i: (0, 0)),
        ],
        out_specs=pl.BlockSpec((_BLK, hdim), lambda i: (i, 0)),
        out_shape=jax.ShapeDtypeStruct((n, hdim), jnp.float32),
    )(a, h0, pp, pp, prev, w, b.reshape(1, hdim))


def _layer0(h0, w, b, *, beta):
    n, hdim = h0.shape

    def bdy(h0_ref, w_ref, b_ref, o_ref):
        hi = h0_ref[...]
        x = ((1.0 - beta) * hi
             + beta * jnp.dot(hi, w_ref[...], preferred_element_type=jnp.float32)
             + b_ref[...])
        o_ref[...] = jnp.maximum(x, 0.0)

    return pl.pallas_call(
        bdy,
        grid=(n // _BLK,),
        in_specs=[
            pl.BlockSpec((_BLK, hdim), lambda i: (i, 0)),
            pl.BlockSpec((hdim, hdim), lambda i: (0, 0)),
            pl.BlockSpec((1, hdim), lambda i: (0, 0)),
        ],
        out_specs=pl.BlockSpec((_BLK, hdim), lambda i: (i, 0)),
        out_shape=jax.ShapeDtypeStruct((n, hdim), jnp.float32),
    )(h0, w, b.reshape(1, hdim))


def _final(x, w, b):
    n, hdim = x.shape
    c = w.shape[1]

    def bdy(x_ref, w_ref, b_ref, o_ref):
        t = jnp.maximum(x_ref[...], 0.0)
        y = (jnp.dot(t, w_ref[...], preferred_element_type=jnp.float32)
             + b_ref[...])
        m = jnp.max(y, axis=1, keepdims=True)
        lse = m + jnp.log(jnp.sum(jnp.exp(y - m), axis=1, keepdims=True))
        o_ref[...] = y - lse

    return pl.pallas_call(
        bdy,
        grid=(n // _BLK,),
        in_specs=[
            pl.BlockSpec((_BLK, hdim), lambda i: (i, 0)),
            pl.BlockSpec((hdim, c), lambda i: (0, 0)),
            pl.BlockSpec((1, c), lambda i: (0, 0)),
        ],
        out_specs=pl.BlockSpec((_BLK, c), lambda i: (i, 0)),
        out_shape=jax.ShapeDtypeStruct((n, c), jnp.float32),
    )(x, w, b.reshape(1, c))


def kernel(features, edge_index, norm_A, W_fc0, b_fc0, conv_W, conv_b,
           W_fc1, b_fc1, alpha_params):
    n = features.shape[0]
    e = norm_A.shape[0]
    hdim = W_fc0.shape[1]
    lnum = conv_W.shape[0] - 1

    src2 = edge_index[0].reshape(e // _WIN, _WIN)
    dst2 = edge_index[1].reshape(e // _WIN, _WIN)
    norm2 = norm_A.reshape(e // _WIN, _WIN)
    zer = jnp.zeros((n // _NS, hdim), jnp.float32)
    prop = _make_prop(n, e, hdim)

    h0 = _fc0(features, W_fc0, b_fc0)
    x = _layer0(h0, conv_W[0], conv_b[0],
                beta=math.log(_LAMDA / 1.0 + 1.0))
    prev = h0
    last = x
    for i in range(1, lnum + 1):
        pp = prop(last, src2, dst2, norm2, zer)
        a = alpha_params[lnum - i].reshape(1)
        beta = math.log(_LAMDA / (i + 1) + 1.0)
        xi = _layer(a, h0, pp, prev, conv_W[i], conv_b[i],
                    beta=beta, tmul=1.0 if i == 1 else 2.0,
                    pmul=0.0 if i == 1 else 1.0,
                    dorelu=i < lnum - 1)
        prev = last
        last = xi
    return _final(last, W_fc1, b_fc1)


# speedup vs baseline: 4.9590x; 4.9590x over previous
import functools
import math

import jax
import jax.numpy as jnp
from jax import lax
from jax.experimental import pallas as pl
from jax.experimental.pallas import tpu as pltpu
from jax.experimental.pallas import tpu_sc as plsc

_NC = 2
_NS = 16
_NW = _NC * _NS
_LANES = 16
_WIN = 128

_LAMDA = 0.5


@functools.lru_cache(maxsize=None)
def _make_prop(n, e, h):
    assert e % _WIN == 0
    nrows = e // _WIN
    base_rows = nrows // _NW
    rem = nrows % _NW
    rps = (-(-n // _NS) + 7) // 8 * 8
    npad = rps * _NS
    ncol = h // _LANES

    mesh = plsc.VectorSubcoreMesh(core_axis_name="c", subcore_axis_name="s")

    def body(h_hbm, src_hbm, dst_hbm, norm_hbm, zer_hbm, out_hbm,
             acc, src_v, dst_v, norm_v, rows_v):
        cid = lax.axis_index("c")
        sid = lax.axis_index("s")
        wid = sid * _NC + cid

        pltpu.sync_copy(zer_hbm, acc.at[pl.ds(sid * rps, rps)])
        plsc.subcore_barrier()

        nwin = jnp.where(wid < rem, base_rows + 1, base_rows)

        def window(j, carry):
            row = wid + _NW * j
            pltpu.sync_copy(src_hbm.at[row], src_v)
            pltpu.sync_copy(dst_hbm.at[row], dst_v)
            pltpu.sync_copy(norm_hbm.at[row], norm_v)
            pltpu.sync_copy(h_hbm.at[src_v], rows_v)

            def scale(g, c2):
                nv16 = norm_v[pl.ds(g * _LANES, _LANES)]
                for l in range(_LANES):
                    vb = jnp.full((_LANES,), nv16[l], jnp.float32)
                    ei = g * _LANES + l
                    for cc in range(ncol):
                        sl = pl.ds(cc * _LANES, _LANES)
                        rows_v[ei, sl] = rows_v[ei, sl] * vb
                return c2

            lax.fori_loop(0, _WIN // _LANES, scale, 0)
            pltpu.sync_copy(rows_v, acc.at[dst_v], add=True)
            return carry

        lax.fori_loop(0, nwin, window, 0)
        plsc.subcore_barrier()
        pltpu.sync_copy(acc.at[pl.ds(sid * rps, rps)],
                        out_hbm.at[cid, pl.ds(sid * rps, rps)])

    return pl.kernel(
        body,
        out_type=jax.ShapeDtypeStruct((2, npad, h), jnp.float32),
        mesh=mesh,
        compiler_params=pltpu.CompilerParams(use_tc_tiling_on_sc=False),
        scratch_types=[
            pltpu.VMEM_SHARED((npad, h), jnp.float32),
            pltpu.VMEM((_WIN,), jnp.int32),
            pltpu.VMEM((_WIN,), jnp.int32),
            pltpu.VMEM((_WIN,), jnp.float32),
            pltpu.VMEM((_WIN, h), jnp.float32),
        ],
    )


_BLK = 400


def _fc0(features, w, b):
    n, din = features.shape
    hdim = w.shape[1]

    def bdy(x_ref, w_ref, b_ref, o_ref):
        o_ref[...] = jnp.maximum(
            jnp.dot(x_ref[...], w_ref[...], preferred_element_type=jnp.float32)
            + b_ref[...], 0.0)

    return pl.pallas_call(
        bdy,
        grid=(n // _BLK,),
        in_specs=[
            pl.BlockSpec((_BLK, din), lambda i: (i, 0)),
            pl.BlockSpec((din, hdim), lambda i: (0, 0)),
            pl.BlockSpec((1, hdim), lambda i: (0, 0)),
        ],
        out_specs=pl.BlockSpec((_BLK, hdim), lambda i: (i, 0)),
        out_shape=jax.ShapeDtypeStruct((n, hdim), jnp.float32),
    )(features, w, b.reshape(1, hdim))


def _layer(a, h0, pp, prev, w, b, *, beta, tmul, pmul, dorelu):
    n, hdim = h0.shape

    def bdy(a_ref, h0_ref, p0_ref, p1_ref, pv_ref, w_ref, b_ref, o_ref):
        av = a_ref[0]
        tx = tmul * (p0_ref[0] + p1_ref[0]) - pmul * pv_ref[...]
        hi = av * h0_ref[...] + (1.0 - av) * tx
        x = ((1.0 - beta) * hi
             + beta * jnp.dot(hi, w_ref[...], preferred_element_type=jnp.float32)
             + b_ref[...])
        o_ref[...] = jnp.maximum(x, 0.0) if dorelu else x

    return pl.pallas_call(
        bdy,
        grid=(n // _BLK,),
        in_specs=[
            pl.BlockSpec(memory_space=pltpu.SMEM),
            pl.BlockSpec((_BLK, hdim), lambda i: (i, 0)),
            pl.BlockSpec((1, _BLK, hdim), lambda i: (0, i, 0)),
            pl.BlockSpec((1, _BLK, hdim), lambda i: (1, i, 0)),
            pl.BlockSpec((_BLK, hdim), lambda i: (i, 0)),
            pl.BlockSpec((hdim, hdim), lambda i: (0, 0)),
            pl.BlockSpec((1, hdim), lambda i: (0, 0)),
        ],
        out_specs=pl.BlockSpec((_BLK, hdim), lambda i: (i, 0)),
        out_shape=jax.ShapeDtypeStruct((n, hdim), jnp.float32),
    )(a, h0, pp, pp, prev, w, b.reshape(1, hdim))


def _layer0(h0, w, b, *, beta):
    n, hdim = h0.shape

    def bdy(h0_ref, w_ref, b_ref, o_ref):
        hi = h0_ref[...]
        x = ((1.0 - beta) * hi
             + beta * jnp.dot(hi, w_ref[...], preferred_element_type=jnp.float32)
             + b_ref[...])
        o_ref[...] = jnp.maximum(x, 0.0)

    return pl.pallas_call(
        bdy,
        grid=(n // _BLK,),
        in_specs=[
            pl.BlockSpec((_BLK, hdim), lambda i: (i, 0)),
            pl.BlockSpec((hdim, hdim), lambda i: (0, 0)),
            pl.BlockSpec((1, hdim), lambda i: (0, 0)),
        ],
        out_specs=pl.BlockSpec((_BLK, hdim), lambda i: (i, 0)),
        out_shape=jax.ShapeDtypeStruct((n, hdim), jnp.float32),
    )(h0, w, b.reshape(1, hdim))


def _final(x, w, b):
    n, hdim = x.shape
    c = w.shape[1]

    def bdy(x_ref, w_ref, b_ref, o_ref):
        t = jnp.maximum(x_ref[...], 0.0)
        y = (jnp.dot(t, w_ref[...], preferred_element_type=jnp.float32)
             + b_ref[...])
        m = jnp.max(y, axis=1, keepdims=True)
        lse = m + jnp.log(jnp.sum(jnp.exp(y - m), axis=1, keepdims=True))
        o_ref[...] = y - lse

    return pl.pallas_call(
        bdy,
        grid=(n // _BLK,),
        in_specs=[
            pl.BlockSpec((_BLK, hdim), lambda i: (i, 0)),
            pl.BlockSpec((hdim, c), lambda i: (0, 0)),
            pl.BlockSpec((1, c), lambda i: (0, 0)),
        ],
        out_specs=pl.BlockSpec((_BLK, c), lambda i: (i, 0)),
        out_shape=jax.ShapeDtypeStruct((n, c), jnp.float32),
    )(x, w, b.reshape(1, c))


def kernel(features, edge_index, norm_A, W_fc0, b_fc0, conv_W, conv_b,
           W_fc1, b_fc1, alpha_params):
    n = features.shape[0]
    e = norm_A.shape[0]
    hdim = W_fc0.shape[1]
    lnum = conv_W.shape[0] - 1

    src2 = edge_index[0].reshape(e // _WIN, _WIN)
    dst2 = edge_index[1].reshape(e // _WIN, _WIN)
    norm2 = norm_A.reshape(e // _WIN, _WIN)
    zer = jnp.zeros(((-(-n // _NS) + 7) // 8 * 8, hdim), jnp.float32)
    prop = _make_prop(n, e, hdim)

    h0 = _fc0(features, W_fc0, b_fc0)
    x = _layer0(h0, conv_W[0], conv_b[0],
                beta=math.log(_LAMDA / 1.0 + 1.0))
    prev = h0
    last = x
    for i in range(1, lnum + 1):
        pp = prop(last, src2, dst2, norm2, zer)
        a = alpha_params[lnum - i].reshape(1)
        beta = math.log(_LAMDA / (i + 1) + 1.0)
        xi = _layer(a, h0, pp, prev, conv_W[i], conv_b[i],
                    beta=beta, tmul=1.0 if i == 1 else 2.0,
                    pmul=0.0 if i == 1 else 1.0,
                    dorelu=i < lnum - 1)
        prev = last
        last = xi
    return _final(last, W_fc1, b_fc1)
